# baseline (speedup 1.0000x reference)
import functools

import jax
import jax.numpy as jnp
from jax import lax
from jax.experimental import pallas as pl
from jax.experimental.pallas import tpu as pltpu
from jax.experimental.pallas import tpu_sc as plsc

N = 10000
E = 320000
D = 128
H = 128
C = 40
C_PAD = 64

NC = 2
NS = 16
NW = NC * NS
EK = 40
E_PER_W = E // NW
N_CHUNKS = E_PER_W // EK
R_BIG = 632
R_LAST = N - (NS - 1) * R_BIG
DEG_PAD = 10240
DEG_PER_TILE = DEG_PAD // NS

_MESH = dict(core_axis_name="c", subcore_axis_name="s", num_cores=NC,
             num_subcores=NS)


def _sc_degree(edge4):
    @functools.partial(
        pl.kernel,
        out_type=jax.ShapeDtypeStruct((NC * DEG_PAD,), jnp.float32),
        mesh=plsc.VectorSubcoreMesh(**_MESH),
        scratch_types=[
            pltpu.VMEM((N_CHUNKS, EK), jnp.int32),
            pltpu.VMEM((EK,), jnp.float32),
            pltpu.VMEM((DEG_PER_TILE,), jnp.float32),
            pltpu.VMEM_SHARED((DEG_PAD,), jnp.float32),
            pltpu.SemaphoreType.DMA,
            pltpu.SemaphoreType.DMA,
        ],
    )
    def deg_kernel(dst_hbm, out_hbm, dst_all, ones_v, zbuf, deg_sh,
                   dsem0, dsem1):
        cid = lax.axis_index("c")
        sid = lax.axis_index("s")
        w = cid * NS + sid

        for j in range(EK // 16):
            ones_v[pl.ds(j * 16, 16)] = jnp.ones((16,), jnp.float32)
        for j in range(DEG_PER_TILE // 16):
            zbuf[pl.ds(j * 16, 16)] = jnp.zeros((16,), jnp.float32)
        pltpu.sync_copy(zbuf, deg_sh.at[pl.ds(sid * DEG_PER_TILE,
                                              DEG_PER_TILE)])
        pltpu.sync_copy(dst_hbm.at[1, w], dst_all)
        plsc.subcore_barrier()

        def s_issue(k, sem):
            pltpu.async_copy(ones_v, deg_sh.at[dst_all.at[k]], sem, add=True)

        def s_wait(k, sem):
            pltpu.make_async_copy(ones_v, deg_sh.at[dst_all.at[k]],
                                  sem).wait()

        s_issue(0, dsem0)

        def body(j, carry):
            k0 = 2 * j
            s_issue(k0 + 1, dsem1)
            s_wait(k0, dsem0)
            s_issue(k0 + 2, dsem0)
            s_wait(k0 + 1, dsem1)
            return carry

        lax.fori_loop(0, (N_CHUNKS - 1) // 2, body, 0)
        s_wait(N_CHUNKS - 1, dsem0)
        plsc.subcore_barrier()
        pltpu.sync_copy(
            deg_sh.at[pl.ds(sid * DEG_PER_TILE, DEG_PER_TILE)],
            out_hbm.at[pl.ds(cid * DEG_PAD + sid * DEG_PER_TILE,
                             DEG_PER_TILE)])

    return deg_kernel(edge4).reshape(NC, DEG_PAD)


def _sc_aggregate(y, edge4, f):
    nbuf = 6 if f >= 128 else 12
    tail = N_CHUNKS % nbuf or nbuf
    n_rounds = (N_CHUNKS - tail) // nbuf

    @functools.partial(
        pl.kernel,
        out_type=jax.ShapeDtypeStruct((NC * N, f), jnp.float32),
        mesh=plsc.VectorSubcoreMesh(**_MESH),
        compiler_params=pltpu.CompilerParams(use_tc_tiling_on_sc=False),
        scratch_types=(
            [pltpu.VMEM((N_CHUNKS, EK), jnp.int32)] * 2
            + [pltpu.VMEM((EK, f), jnp.float32)
               for _ in range(nbuf)]
            + [pltpu.VMEM_SHARED((N, f), jnp.float32)]
            + [pltpu.SemaphoreType.DMA] * (2 * nbuf)
        ),
    )
    def agg_kernel(y_hbm, edge_hbm, out_hbm, src_all, dst_all, *bufs):
        rows = list(bufs[:nbuf])
        agg_sh = bufs[nbuf]
        gsem = list(bufs[nbuf + 1:2 * nbuf + 1])
        ssem = list(bufs[2 * nbuf + 1:3 * nbuf + 1])
        cid = lax.axis_index("c")
        sid = lax.axis_index("s")
        w = cid * NS + sid
        r0 = sid * R_BIG

        pltpu.sync_copy(edge_hbm.at[0, w], src_all)
        pltpu.sync_copy(edge_hbm.at[1, w], dst_all)

        def g_issue(k, s):
            pltpu.async_copy(y_hbm.at[src_all.at[k]], rows[s], gsem[s])

        def g_wait(k, s):
            pltpu.make_async_copy(y_hbm.at[src_all.at[k]], rows[s],
                                  gsem[s]).wait()

        def s_issue(k, s):
            pltpu.async_copy(rows[s], agg_sh.at[dst_all.at[k]], ssem[s],
                             add=True)

        def s_wait(k, s):
            pltpu.make_async_copy(rows[s], agg_sh.at[dst_all.at[k]],
                                  ssem[s]).wait()

        for s in range(nbuf):
            g_issue(s, s)

        @pl.when(sid < NS - 1)
        def _():
            pltpu.sync_copy(y_hbm.at[pl.ds(r0, R_BIG)],
                            agg_sh.at[pl.ds(r0, R_BIG)])

        @pl.when(sid == NS - 1)
        def _():
            pltpu.sync_copy(y_hbm.at[pl.ds((NS - 1) * R_BIG, R_LAST)],
                            agg_sh.at[pl.ds((NS - 1) * R_BIG, R_LAST)])

        plsc.subcore_barrier()

        def body(j, carry):
            k0 = nbuf * j
            for s in range(nbuf):
                g_wait(k0 + s, s)
                s_issue(k0 + s, s)
            for s in range(nbuf):
                s_wait(k0 + s, s)
                if s < tail:
                    g_issue(k0 + nbuf + s, s)
                else:
                    @pl.when(j < n_rounds - 1)
                    def _():
                        g_issue(k0 + nbuf + s, s)
            return carry

        lax.fori_loop(0, n_rounds, body, 0)
        for t in range(tail):
            k = N_CHUNKS - tail + t
            g_wait(k, t)
            pltpu.sync_copy(rows[t], agg_sh.at[dst_all.at[k]], add=True)
        plsc.subcore_barrier()

        @pl.when(sid < NS - 1)
        def _():
            pltpu.sync_copy(agg_sh.at[pl.ds(r0, R_BIG)],
                            out_hbm.at[pl.ds(cid * N + r0, R_BIG)])

        @pl.when(sid == NS - 1)
        def _():
            pltpu.sync_copy(
                agg_sh.at[pl.ds((NS - 1) * R_BIG, R_LAST)],
                out_hbm.at[pl.ds(cid * N + (NS - 1) * R_BIG, R_LAST)])

    return agg_kernel(y, edge4)


_BLK = 2000
_GRID = N // _BLK


def _dinv_block(d_ref):
    return lax.rsqrt(d_ref[:, 0:1] + d_ref[:, 1:2] + 1.0)


def _tc_scale(x, deg_col):
    def body(d_ref, x_ref, u_ref):
        u_ref[...] = x_ref[...] * _dinv_block(d_ref)

    return pl.pallas_call(
        body,
        grid=(_GRID,),
        in_specs=[
            pl.BlockSpec((_BLK, 2), lambda i: (i, 0)),
            pl.BlockSpec((_BLK, D), lambda i: (i, 0)),
        ],
        out_specs=pl.BlockSpec((_BLK, D), lambda i: (i, 0)),
        out_shape=jax.ShapeDtypeStruct((N, D), jnp.float32),
    )(deg_col, x)


def _tc_mid(agg2, u, deg_col, w1, b1_row, w2p):
    def body(a0_ref, a1_ref, u_ref, d_ref, w1_ref, b1_ref, w2_ref, out_ref):
        dinv = _dinv_block(d_ref)
        v = a0_ref[...] + a1_ref[...] - u_ref[...]
        t = jnp.dot(v, w1_ref[...],
                    preferred_element_type=jnp.float32) * dinv + b1_ref[...]
        h = jnp.maximum(t, 0.0)
        out_ref[...] = jnp.dot(h, w2_ref[...],
                               preferred_element_type=jnp.float32) * dinv

    f_out = w2p.shape[1]
    flat = agg2
    return pl.pallas_call(
        body,
        grid=(_GRID,),
        in_specs=[
            pl.BlockSpec((_BLK, D), lambda i: (i, 0)),
            pl.BlockSpec((_BLK, D), lambda i: (i + _GRID, 0)),
            pl.BlockSpec((_BLK, D), lambda i: (i, 0)),
            pl.BlockSpec((_BLK, 2), lambda i: (i, 0)),
            pl.BlockSpec((D, D), lambda i: (0, 0)),
            pl.BlockSpec((1, D), lambda i: (0, 0)),
            pl.BlockSpec((D, f_out), lambda i: (0, 0)),
        ],
        out_specs=pl.BlockSpec((_BLK, f_out), lambda i: (i, 0)),
        out_shape=jax.ShapeDtypeStruct((N, f_out), jnp.float32),
    )(flat, flat, u, deg_col, w1, b1_row, w2p)


def _tc_combine_final(agg2, y, deg_col, b_row):
    def body(a0_ref, a1_ref, y_ref, d_ref, b_ref, out_ref):
        out_ref[...] = ((a0_ref[...] + a1_ref[...] - y_ref[...])
                        * _dinv_block(d_ref) + b_ref[...])

    f = y.shape[1]
    flat = agg2
    return pl.pallas_call(
        body,
        grid=(_GRID,),
        in_specs=[
            pl.BlockSpec((_BLK, f), lambda i: (i, 0)),
            pl.BlockSpec((_BLK, f), lambda i: (i + _GRID, 0)),
            pl.BlockSpec((_BLK, f), lambda i: (i, 0)),
            pl.BlockSpec((_BLK, 2), lambda i: (i, 0)),
            pl.BlockSpec((1, f), lambda i: (0, 0)),
        ],
        out_specs=pl.BlockSpec((_BLK, f), lambda i: (i, 0)),
        out_shape=jax.ShapeDtypeStruct((N, f), jnp.float32),
    )(flat, flat, y, deg_col, b_row)


def kernel(x, edge_index, W1, b1, W2, b2):
    edge4 = edge_index.reshape(2, NW, N_CHUNKS, EK)

    deg2 = _sc_degree(edge4)
    deg_col = deg2.T

    u = _tc_scale(x, deg_col)
    agg1 = _sc_aggregate(u, edge4, D)

    y2 = _tc_mid(agg1, u, deg_col, W1, b1.reshape(1, H), W2)
    agg2 = _sc_aggregate(y2, edge4, C)

    return _tc_combine_final(agg2, y2, deg_col, b2.reshape(1, C))


# speedup vs baseline: 1.0227x; 1.0227x over previous
import functools

import jax
import jax.numpy as jnp
from jax import lax
from jax.experimental import pallas as pl
from jax.experimental.pallas import tpu as pltpu
from jax.experimental.pallas import tpu_sc as plsc

N = 10000
E = 320000
D = 128
H = 128
C = 40
C_PAD = 64

NC = 2
NS = 16
NW = NC * NS
EK = 80
E_PER_W = E // NW
N_CHUNKS = E_PER_W // EK
R_BIG = 632
R_LAST = N - (NS - 1) * R_BIG
DEG_PAD = 10240
DEG_PER_TILE = DEG_PAD // NS

_MESH = dict(core_axis_name="c", subcore_axis_name="s", num_cores=NC,
             num_subcores=NS)


def _sc_degree(edge4):
    @functools.partial(
        pl.kernel,
        out_type=jax.ShapeDtypeStruct((NC * DEG_PAD,), jnp.float32),
        mesh=plsc.VectorSubcoreMesh(**_MESH),
        scratch_types=[
            pltpu.VMEM((N_CHUNKS, EK), jnp.int32),
            pltpu.VMEM((EK,), jnp.float32),
            pltpu.VMEM((DEG_PER_TILE,), jnp.float32),
            pltpu.VMEM_SHARED((DEG_PAD,), jnp.float32),
            pltpu.SemaphoreType.DMA,
            pltpu.SemaphoreType.DMA,
        ],
    )
    def deg_kernel(dst_hbm, out_hbm, dst_all, ones_v, zbuf, deg_sh,
                   dsem0, dsem1):
        cid = lax.axis_index("c")
        sid = lax.axis_index("s")
        w = cid * NS + sid

        for j in range(EK // 16):
            ones_v[pl.ds(j * 16, 16)] = jnp.ones((16,), jnp.float32)
        for j in range(DEG_PER_TILE // 16):
            zbuf[pl.ds(j * 16, 16)] = jnp.zeros((16,), jnp.float32)
        pltpu.sync_copy(zbuf, deg_sh.at[pl.ds(sid * DEG_PER_TILE,
                                              DEG_PER_TILE)])
        pltpu.sync_copy(dst_hbm.at[1, w], dst_all)
        plsc.subcore_barrier()

        def s_issue(k, sem):
            pltpu.async_copy(ones_v, deg_sh.at[dst_all.at[k]], sem, add=True)

        def s_wait(k, sem):
            pltpu.make_async_copy(ones_v, deg_sh.at[dst_all.at[k]],
                                  sem).wait()

        s_issue(0, dsem0)

        def body(j, carry):
            k0 = 2 * j
            s_issue(k0 + 1, dsem1)
            s_wait(k0, dsem0)
            s_issue(k0 + 2, dsem0)
            s_wait(k0 + 1, dsem1)
            return carry

        lax.fori_loop(0, (N_CHUNKS - 1) // 2, body, 0)
        s_wait(N_CHUNKS - 1, dsem0)
        plsc.subcore_barrier()
        pltpu.sync_copy(
            deg_sh.at[pl.ds(sid * DEG_PER_TILE, DEG_PER_TILE)],
            out_hbm.at[pl.ds(cid * DEG_PAD + sid * DEG_PER_TILE,
                             DEG_PER_TILE)])

    return deg_kernel(edge4).reshape(NC, DEG_PAD)


def _sc_aggregate(y, edge4, f):
    nbuf = 3 if f >= 128 else 8
    tail = N_CHUNKS % nbuf or nbuf
    n_rounds = (N_CHUNKS - tail) // nbuf

    @functools.partial(
        pl.kernel,
        out_type=jax.ShapeDtypeStruct((NC * N, f), jnp.float32),
        mesh=plsc.VectorSubcoreMesh(**_MESH),
        compiler_params=pltpu.CompilerParams(use_tc_tiling_on_sc=False),
        scratch_types=(
            [pltpu.VMEM((N_CHUNKS, EK), jnp.int32)] * 2
            + [pltpu.VMEM((EK, f), jnp.float32)
               for _ in range(nbuf)]
            + [pltpu.VMEM_SHARED((N, f), jnp.float32)]
            + [pltpu.SemaphoreType.DMA] * (2 * nbuf)
        ),
    )
    def agg_kernel(y_hbm, edge_hbm, out_hbm, src_all, dst_all, *bufs):
        rows = list(bufs[:nbuf])
        agg_sh = bufs[nbuf]
        gsem = list(bufs[nbuf + 1:2 * nbuf + 1])
        ssem = list(bufs[2 * nbuf + 1:3 * nbuf + 1])
        cid = lax.axis_index("c")
        sid = lax.axis_index("s")
        w = cid * NS + sid
        r0 = sid * R_BIG

        pltpu.sync_copy(edge_hbm.at[0, w], src_all)
        pltpu.sync_copy(edge_hbm.at[1, w], dst_all)

        def g_issue(k, s):
            pltpu.async_copy(y_hbm.at[src_all.at[k]], rows[s], gsem[s])

        def g_wait(k, s):
            pltpu.make_async_copy(y_hbm.at[src_all.at[k]], rows[s],
                                  gsem[s]).wait()

        def s_issue(k, s):
            pltpu.async_copy(rows[s], agg_sh.at[dst_all.at[k]], ssem[s],
                             add=True)

        def s_wait(k, s):
            pltpu.make_async_copy(rows[s], agg_sh.at[dst_all.at[k]],
                                  ssem[s]).wait()

        for s in range(nbuf):
            g_issue(s, s)

        @pl.when(sid < NS - 1)
        def _():
            pltpu.sync_copy(y_hbm.at[pl.ds(r0, R_BIG)],
                            agg_sh.at[pl.ds(r0, R_BIG)])

        @pl.when(sid == NS - 1)
        def _():
            pltpu.sync_copy(y_hbm.at[pl.ds((NS - 1) * R_BIG, R_LAST)],
                            agg_sh.at[pl.ds((NS - 1) * R_BIG, R_LAST)])

        plsc.subcore_barrier()

        def body(j, carry):
            k0 = nbuf * j
            for s in range(nbuf):
                g_wait(k0 + s, s)
                s_issue(k0 + s, s)
            for s in range(nbuf):
                s_wait(k0 + s, s)
                if s < tail:
                    g_issue(k0 + nbuf + s, s)
                else:
                    @pl.when(j < n_rounds - 1)
                    def _():
                        g_issue(k0 + nbuf + s, s)
            return carry

        lax.fori_loop(0, n_rounds, body, 0)
        for t in range(tail):
            k = N_CHUNKS - tail + t
            g_wait(k, t)
            pltpu.sync_copy(rows[t], agg_sh.at[dst_all.at[k]], add=True)
        plsc.subcore_barrier()

        @pl.when(sid < NS - 1)
        def _():
            pltpu.sync_copy(agg_sh.at[pl.ds(r0, R_BIG)],
                            out_hbm.at[pl.ds(cid * N + r0, R_BIG)])

        @pl.when(sid == NS - 1)
        def _():
            pltpu.sync_copy(
                agg_sh.at[pl.ds((NS - 1) * R_BIG, R_LAST)],
                out_hbm.at[pl.ds(cid * N + (NS - 1) * R_BIG, R_LAST)])

    return agg_kernel(y, edge4)


_BLK = 2000
_GRID = N // _BLK


def _dinv_block(d_ref):
    return lax.rsqrt(d_ref[:, 0:1] + d_ref[:, 1:2] + 1.0)


def _tc_scale(x, deg_col):
    def body(d_ref, x_ref, u_ref):
        u_ref[...] = x_ref[...] * _dinv_block(d_ref)

    return pl.pallas_call(
        body,
        grid=(_GRID,),
        in_specs=[
            pl.BlockSpec((_BLK, 2), lambda i: (i, 0)),
            pl.BlockSpec((_BLK, D), lambda i: (i, 0)),
        ],
        out_specs=pl.BlockSpec((_BLK, D), lambda i: (i, 0)),
        out_shape=jax.ShapeDtypeStruct((N, D), jnp.float32),
    )(deg_col, x)


def _tc_mid(agg2, u, deg_col, w1, b1_row, w2p):
    def body(a0_ref, a1_ref, u_ref, d_ref, w1_ref, b1_ref, w2_ref, out_ref):
        dinv = _dinv_block(d_ref)
        v = a0_ref[...] + a1_ref[...] - u_ref[...]
        t = jnp.dot(v, w1_ref[...],
                    preferred_element_type=jnp.float32) * dinv + b1_ref[...]
        h = jnp.maximum(t, 0.0)
        out_ref[...] = jnp.dot(h, w2_ref[...],
                               preferred_element_type=jnp.float32) * dinv

    f_out = w2p.shape[1]
    flat = agg2
    return pl.pallas_call(
        body,
        grid=(_GRID,),
        in_specs=[
            pl.BlockSpec((_BLK, D), lambda i: (i, 0)),
            pl.BlockSpec((_BLK, D), lambda i: (i + _GRID, 0)),
            pl.BlockSpec((_BLK, D), lambda i: (i, 0)),
            pl.BlockSpec((_BLK, 2), lambda i: (i, 0)),
            pl.BlockSpec((D, D), lambda i: (0, 0)),
            pl.BlockSpec((1, D), lambda i: (0, 0)),
            pl.BlockSpec((D, f_out), lambda i: (0, 0)),
        ],
        out_specs=pl.BlockSpec((_BLK, f_out), lambda i: (i, 0)),
        out_shape=jax.ShapeDtypeStruct((N, f_out), jnp.float32),
    )(flat, flat, u, deg_col, w1, b1_row, w2p)


def _tc_combine_final(agg2, y, deg_col, b_row):
    def body(a0_ref, a1_ref, y_ref, d_ref, b_ref, out_ref):
        out_ref[...] = ((a0_ref[...] + a1_ref[...] - y_ref[...])
                        * _dinv_block(d_ref) + b_ref[...])

    f = y.shape[1]
    flat = agg2
    return pl.pallas_call(
        body,
        grid=(_GRID,),
        in_specs=[
            pl.BlockSpec((_BLK, f), lambda i: (i, 0)),
            pl.BlockSpec((_BLK, f), lambda i: (i + _GRID, 0)),
            pl.BlockSpec((_BLK, f), lambda i: (i, 0)),
            pl.BlockSpec((_BLK, 2), lambda i: (i, 0)),
            pl.BlockSpec((1, f), lambda i: (0, 0)),
        ],
        out_specs=pl.BlockSpec((_BLK, f), lambda i: (i, 0)),
        out_shape=jax.ShapeDtypeStruct((N, f), jnp.float32),
    )(flat, flat, y, deg_col, b_row)


def kernel(x, edge_index, W1, b1, W2, b2):
    edge4 = edge_index.reshape(2, NW, N_CHUNKS, EK)

    deg2 = _sc_degree(edge4)
    deg_col = deg2.T

    u = _tc_scale(x, deg_col)
    agg1 = _sc_aggregate(u, edge4, D)

    y2 = _tc_mid(agg1, u, deg_col, W1, b1.reshape(1, H), W2)
    agg2 = _sc_aggregate(y2, edge4, C)

    return _tc_combine_final(agg2, y2, deg_col, b2.reshape(1, C))
